# initial kernel scaffold (unmeasured)
import jax
import jax.numpy as jnp
from jax import lax
from jax.experimental import pallas as pl
from jax.experimental.pallas import tpu as pltpu

N_DEV = 16


def kernel(x, w_mat, scale_x, scale_w):
    m_per, k = x.shape
    k2, n_per = w_mat.shape
    assert k == k2

    def body(x_ref, w_ref, sx_ref, sw_ref, out_ref, comm_ref, send_sems, recv_sems):
        my = lax.axis_index("i")
        left = lax.rem(my + N_DEV - 1, N_DEV)
        right = lax.rem(my + 1, N_DEV)

        barrier_sem = pltpu.get_barrier_semaphore()
        for nbr in (left, right):
            pl.semaphore_signal(
                barrier_sem, inc=1,
                device_id=(nbr,), device_id_type=pl.DeviceIdType.MESH,
            )
        pl.semaphore_wait(barrier_sem, 2)

        scale = sx_ref[0] * sw_ref[0]

        comm_ref[0] = x_ref[...]

        acc = jnp.dot(x_ref[...], w_ref[...], preferred_element_type=jnp.float32)
        out_ref[pl.ds(my * m_per, m_per), :] = acc * scale

        for h in range(N_DEV - 1):
            rdma = pltpu.make_async_remote_copy(
                src_ref=comm_ref.at[h],
                dst_ref=comm_ref.at[h + 1],
                send_sem=send_sems.at[h],
                recv_sem=recv_sems.at[h],
                device_id=(right,),
                device_id_type=pl.DeviceIdType.MESH,
            )
            rdma.start()
            rdma.wait()

            origin = lax.rem(my + N_DEV - 1 - h, N_DEV)
            acc = jnp.dot(
                comm_ref[h + 1], w_ref[...], preferred_element_type=jnp.float32
            )
            out_ref[pl.ds(origin * m_per, m_per), :] = acc * scale

    return pl.pallas_call(
        body,
        out_shape=jax.ShapeDtypeStruct((N_DEV * m_per, n_per), jnp.float32),
        in_specs=[
            pl.BlockSpec(memory_space=pltpu.VMEM),
            pl.BlockSpec(memory_space=pltpu.VMEM),
            pl.BlockSpec(memory_space=pltpu.SMEM),
            pl.BlockSpec(memory_space=pltpu.SMEM),
        ],
        out_specs=pl.BlockSpec(memory_space=pltpu.VMEM),
        scratch_shapes=[
            pltpu.VMEM((N_DEV, m_per, k), x.dtype),
            pltpu.SemaphoreType.DMA((N_DEV - 1,)),
            pltpu.SemaphoreType.DMA((N_DEV - 1,)),
        ],
        compiler_params=pltpu.CompilerParams(collective_id=0),
    )(x, w_mat, scale_x, scale_w)


# baseline (device time: 221466 ns/iter reference)
import jax
import jax.numpy as jnp
from jax import lax
from jax.experimental import pallas as pl
from jax.experimental.pallas import tpu as pltpu

N_DEV = 16


def kernel(x, w_mat, scale_x, scale_w):
    m_per, k = x.shape
    k2, n_per = w_mat.shape
    assert k == k2

    def body(x_ref, w_ref, sx_ref, sw_ref, out_ref, comm_ref, w8_ref, send_sems, recv_sems):
        my = lax.axis_index("i")
        left = lax.rem(my + N_DEV - 1, N_DEV)
        right = lax.rem(my + 1, N_DEV)

        barrier_sem = pltpu.get_barrier_semaphore()
        for nbr in (left, right):
            pl.semaphore_signal(
                barrier_sem, inc=1,
                device_id=(nbr,), device_id_type=pl.DeviceIdType.MESH,
            )
        pl.semaphore_wait(barrier_sem, 2)

        scale = sx_ref[0] * sw_ref[0]

        w8_ref[...] = w_ref[...].astype(jnp.float8_e4m3fn)
        comm_ref[0] = x_ref[...].astype(jnp.float8_e4m3fn)

        acc = jnp.dot(comm_ref[0], w8_ref[...], preferred_element_type=jnp.float32)
        out_ref[pl.ds(my * m_per, m_per), :] = acc * scale

        for h in range(N_DEV - 1):
            rdma = pltpu.make_async_remote_copy(
                src_ref=comm_ref.at[h],
                dst_ref=comm_ref.at[h + 1],
                send_sem=send_sems.at[h],
                recv_sem=recv_sems.at[h],
                device_id=(right,),
                device_id_type=pl.DeviceIdType.MESH,
            )
            rdma.start()
            rdma.wait()

            origin = lax.rem(my + N_DEV - 1 - h, N_DEV)
            acc = jnp.dot(
                comm_ref[h + 1], w8_ref[...], preferred_element_type=jnp.float32
            )
            out_ref[pl.ds(origin * m_per, m_per), :] = acc * scale

    return pl.pallas_call(
        body,
        out_shape=jax.ShapeDtypeStruct((N_DEV * m_per, n_per), jnp.float32),
        in_specs=[
            pl.BlockSpec(memory_space=pltpu.VMEM),
            pl.BlockSpec(memory_space=pltpu.VMEM),
            pl.BlockSpec(memory_space=pltpu.SMEM),
            pl.BlockSpec(memory_space=pltpu.SMEM),
        ],
        out_specs=pl.BlockSpec(memory_space=pltpu.VMEM),
        scratch_shapes=[
            pltpu.VMEM((N_DEV, m_per, k), jnp.float8_e4m3fn),
            pltpu.VMEM((k, n_per), jnp.float8_e4m3fn),
            pltpu.SemaphoreType.DMA((N_DEV - 1,)),
            pltpu.SemaphoreType.DMA((N_DEV - 1,)),
        ],
        compiler_params=pltpu.CompilerParams(collective_id=0),
    )(x, w_mat, scale_x, scale_w)


# device time: 118520 ns/iter; 1.8686x vs baseline; 1.8686x over previous
import jax
import jax.numpy as jnp
from jax import lax
from jax.experimental import pallas as pl
from jax.experimental.pallas import tpu as pltpu

N_DEV = 16
N_FWD = 8
N_BWD = 7


def kernel(x, w_mat, scale_x, scale_w):
    m_per, k = x.shape
    k2, n_per = w_mat.shape
    assert k == k2

    def body(x_ref, w_ref, sx_ref, sw_ref, out_ref,
             fwd_ref, bwd_ref, w8_ref,
             fs_sems, fr_sems, bs_sems, br_sems):
        my = lax.axis_index("i")
        left = lax.rem(my + N_DEV - 1, N_DEV)
        right = lax.rem(my + 1, N_DEV)

        barrier_sem = pltpu.get_barrier_semaphore()
        for nbr in (left, right):
            pl.semaphore_signal(
                barrier_sem, inc=1,
                device_id=(nbr,), device_id_type=pl.DeviceIdType.MESH,
            )
        pl.semaphore_wait(barrier_sem, 2)

        scale = sx_ref[0] * sw_ref[0]

        x8 = x_ref[...].astype(jnp.float8_e4m3fn)
        fwd_ref[0] = x8
        bwd_ref[0] = x8

        def fwd_rdma(h):
            return pltpu.make_async_remote_copy(
                src_ref=fwd_ref.at[h],
                dst_ref=fwd_ref.at[h + 1],
                send_sem=fs_sems.at[h],
                recv_sem=fr_sems.at[h],
                device_id=(right,),
                device_id_type=pl.DeviceIdType.MESH,
            )

        def bwd_rdma(h):
            return pltpu.make_async_remote_copy(
                src_ref=bwd_ref.at[h],
                dst_ref=bwd_ref.at[h + 1],
                send_sem=bs_sems.at[h],
                recv_sem=br_sems.at[h],
                device_id=(left,),
                device_id_type=pl.DeviceIdType.MESH,
            )

        fwd_rdmas = [fwd_rdma(h) for h in range(N_FWD)]
        bwd_rdmas = [bwd_rdma(h) for h in range(N_BWD)]

        fwd_rdmas[0].start()
        bwd_rdmas[0].start()

        w8_ref[...] = w_ref[...].astype(jnp.float8_e4m3fn)
        acc = jnp.dot(fwd_ref[0], w8_ref[...], preferred_element_type=jnp.float32)
        out_ref[pl.ds(my * m_per, m_per), :] = acc * scale

        for h in range(N_FWD):
            fwd_rdmas[h].wait_recv()
            if h + 1 < N_FWD:
                fwd_rdmas[h + 1].start()
            if h < N_BWD:
                bwd_rdmas[h].wait_recv()
                if h + 1 < N_BWD:
                    bwd_rdmas[h + 1].start()

            origin_f = lax.rem(my + N_DEV - 1 - h, N_DEV)
            acc = jnp.dot(
                fwd_ref[h + 1], w8_ref[...], preferred_element_type=jnp.float32
            )
            out_ref[pl.ds(origin_f * m_per, m_per), :] = acc * scale

            if h < N_BWD:
                origin_b = lax.rem(my + 1 + h, N_DEV)
                acc = jnp.dot(
                    bwd_ref[h + 1], w8_ref[...], preferred_element_type=jnp.float32
                )
                out_ref[pl.ds(origin_b * m_per, m_per), :] = acc * scale

        for r in fwd_rdmas:
            r.wait_send()
        for r in bwd_rdmas:
            r.wait_send()

    return pl.pallas_call(
        body,
        out_shape=jax.ShapeDtypeStruct((N_DEV * m_per, n_per), jnp.float32),
        in_specs=[
            pl.BlockSpec(memory_space=pltpu.VMEM),
            pl.BlockSpec(memory_space=pltpu.VMEM),
            pl.BlockSpec(memory_space=pltpu.SMEM),
            pl.BlockSpec(memory_space=pltpu.SMEM),
        ],
        out_specs=pl.BlockSpec(memory_space=pltpu.VMEM),
        scratch_shapes=[
            pltpu.VMEM((N_FWD + 1, m_per, k), jnp.float8_e4m3fn),
            pltpu.VMEM((N_BWD + 1, m_per, k), jnp.float8_e4m3fn),
            pltpu.VMEM((k, n_per), jnp.float8_e4m3fn),
            pltpu.SemaphoreType.DMA((N_FWD,)),
            pltpu.SemaphoreType.DMA((N_FWD,)),
            pltpu.SemaphoreType.DMA((N_BWD,)),
            pltpu.SemaphoreType.DMA((N_BWD,)),
        ],
        compiler_params=pltpu.CompilerParams(collective_id=0),
    )(x, w_mat, scale_x, scale_w)


# device time: 102207 ns/iter; 2.1668x vs baseline; 1.1596x over previous
import jax
import jax.numpy as jnp
from jax import lax
from jax.experimental import pallas as pl
from jax.experimental.pallas import tpu as pltpu

N_DEV = 16
N_MSG = 15
M_HALF = 128


def kernel(x, w_mat, scale_x, scale_w):
    m_per, k = x.shape
    k2, n_per = w_mat.shape
    assert k == k2 and m_per == 2 * M_HALF

    def body(x_ref, w_ref, sx_ref, sw_ref, out_ref,
             fwd_ref, bwd_ref, w8_ref,
             fs_sems, fr_sems, bs_sems, br_sems):
        my = lax.axis_index("i")
        left = lax.rem(my + N_DEV - 1, N_DEV)
        right = lax.rem(my + 1, N_DEV)

        barrier_sem = pltpu.get_barrier_semaphore()
        for nbr in (left, right):
            pl.semaphore_signal(
                barrier_sem, inc=1,
                device_id=(nbr,), device_id_type=pl.DeviceIdType.MESH,
            )
        pl.semaphore_wait(barrier_sem, 2)

        scale = sx_ref[0] * sw_ref[0]

        x8 = x_ref[...].astype(jnp.float8_e4m3fn)
        fwd_ref[pl.ds(0, M_HALF), :] = x8[:M_HALF]
        fwd_ref[pl.ds(M_HALF, M_HALF), :] = x8[M_HALF:]
        bwd_ref[pl.ds(0, M_HALF), :] = x8[M_HALF:]
        bwd_ref[pl.ds(M_HALF, M_HALF), :] = x8[:M_HALF]

        def mk(buf_ref, s, ssems, rsems, dev):
            return pltpu.make_async_remote_copy(
                src_ref=buf_ref.at[pl.ds(s * M_HALF, M_HALF), :],
                dst_ref=buf_ref.at[pl.ds((s + 2) * M_HALF, M_HALF), :],
                send_sem=ssems.at[s],
                recv_sem=rsems.at[s],
                device_id=(dev,),
                device_id_type=pl.DeviceIdType.MESH,
            )

        fwd_rdmas = [mk(fwd_ref, s, fs_sems, fr_sems, right) for s in range(N_MSG)]
        bwd_rdmas = [mk(bwd_ref, s, bs_sems, br_sems, left) for s in range(N_MSG)]

        fwd_rdmas[0].start()
        bwd_rdmas[0].start()
        fwd_rdmas[1].start()
        bwd_rdmas[1].start()

        w8_ref[...] = w_ref[...].astype(jnp.float8_e4m3fn)
        acc = jnp.dot(x8, w8_ref[...], preferred_element_type=jnp.float32)
        out_ref[pl.ds(my * m_per, m_per), :] = acc * scale

        for s in range(N_MSG):
            fwd_rdmas[s].wait_recv()
            if s + 2 < N_MSG:
                fwd_rdmas[s + 2].start()
            bwd_rdmas[s].wait_recv()
            if s + 2 < N_MSG:
                bwd_rdmas[s + 2].start()

            if s % 2 == 1:
                j = (s - 1) // 2
                origin_f = lax.rem(my + N_DEV - (j + 1), N_DEV)
                chunk_f = fwd_ref[pl.ds((s + 1) * M_HALF, m_per), :]
                acc = jnp.dot(chunk_f, w8_ref[...],
                              preferred_element_type=jnp.float32)
                out_ref[pl.ds(origin_f * m_per, m_per), :] = acc * scale

                origin_b = lax.rem(my + j + 1, N_DEV)
                chunk_b = bwd_ref[pl.ds((s + 1) * M_HALF, m_per), :]
                acc = jnp.dot(chunk_b, w8_ref[...],
                              preferred_element_type=jnp.float32)
                out_ref[pl.ds(origin_b * m_per + M_HALF, M_HALF), :] = \
                    acc[:M_HALF] * scale
                out_ref[pl.ds(origin_b * m_per, M_HALF), :] = \
                    acc[M_HALF:] * scale

        origin8 = lax.rem(my + N_DEV // 2, N_DEV)
        acc = jnp.dot(fwd_ref[pl.ds(16 * M_HALF, M_HALF), :], w8_ref[...],
                      preferred_element_type=jnp.float32)
        out_ref[pl.ds(origin8 * m_per, M_HALF), :] = acc * scale
        acc = jnp.dot(bwd_ref[pl.ds(16 * M_HALF, M_HALF), :], w8_ref[...],
                      preferred_element_type=jnp.float32)
        out_ref[pl.ds(origin8 * m_per + M_HALF, M_HALF), :] = acc * scale

        for r in fwd_rdmas:
            r.wait_send()
        for r in bwd_rdmas:
            r.wait_send()

    n_slots = N_MSG + 2
    return pl.pallas_call(
        body,
        out_shape=jax.ShapeDtypeStruct((N_DEV * m_per, n_per), jnp.float32),
        in_specs=[
            pl.BlockSpec(memory_space=pltpu.VMEM),
            pl.BlockSpec(memory_space=pltpu.VMEM),
            pl.BlockSpec(memory_space=pltpu.SMEM),
            pl.BlockSpec(memory_space=pltpu.SMEM),
        ],
        out_specs=pl.BlockSpec(memory_space=pltpu.VMEM),
        scratch_shapes=[
            pltpu.VMEM((n_slots * M_HALF, k), jnp.float8_e4m3fn),
            pltpu.VMEM((n_slots * M_HALF, k), jnp.float8_e4m3fn),
            pltpu.VMEM((k, n_per), jnp.float8_e4m3fn),
            pltpu.SemaphoreType.DMA((N_MSG,)),
            pltpu.SemaphoreType.DMA((N_MSG,)),
            pltpu.SemaphoreType.DMA((N_MSG,)),
            pltpu.SemaphoreType.DMA((N_MSG,)),
        ],
        compiler_params=pltpu.CompilerParams(collective_id=0),
    )(x, w_mat, scale_x, scale_w)
